# Initial kernel scaffold; baseline (speedup 1.0000x reference)
import jax
import jax.numpy as jnp
from jax.experimental import pallas as pl


def kernel(bbox, x_table, y_table, h_table, w_table):
    raise NotImplementedError("write your pallas kernel here")


# speedup vs baseline: 7.6186x; 7.6186x over previous
import functools

import jax
import jax.numpy as jnp
from jax import lax
from jax.experimental import pallas as pl
from jax.experimental.pallas import tpu as pltpu
from jax.experimental.pallas import tpu_sc as plsc

B = 1024
N = 200
COORD = 128
T = B * N
D_OUT = 6 * COORD
NW = 32
TPW = T // NW
C = 128
NCHUNK = TPW // C


def _make_sc_kernel():
    mesh = plsc.VectorSubcoreMesh(core_axis_name="c", subcore_axis_name="s")

    @functools.partial(
        pl.kernel,
        out_type=jax.ShapeDtypeStruct((T, D_OUT), jnp.float32),
        mesh=mesh,
        scratch_types=[
            pltpu.VMEM((4, C), jnp.int32),
            [pltpu.VMEM((C,), jnp.int32) for _ in range(6)],
            [pltpu.VMEM((C, COORD), jnp.float32) for _ in range(6)],
            pltpu.SemaphoreType.DMA,
        ],
    )
    def body(bbox_hbm, x_hbm, y_hbm, h_hbm, w_hbm, out_hbm,
             bb_v, idx_v, row_v, sem):
        wid = lax.axis_index("s") * 2 + lax.axis_index("c")
        tables = (x_hbm, y_hbm, x_hbm, y_hbm, h_hbm, w_hbm)

        def chunk(ci, _):
            base = wid * TPW + ci * C
            pltpu.sync_copy(bbox_hbm.at[:, pl.ds(base, C)], bb_v)
            for i in range(C // 16):
                sl = pl.ds(i * 16, 16)
                c0 = bb_v[0, sl]
                c1 = bb_v[1, sl]
                c2 = bb_v[2, sl]
                c3 = bb_v[3, sl]
                idx_v[0][sl] = c0
                idx_v[1][sl] = c1
                idx_v[2][sl] = c2
                idx_v[3][sl] = c3
                idx_v[4][sl] = c3 - c1
                idx_v[5][sl] = c2 - c0
            copies = [
                pltpu.async_copy(tables[g].at[idx_v[g]], row_v[g], sem)
                for g in range(6)
            ]
            for c in copies:
                c.wait()
            for g in range(6):
                pltpu.sync_copy(
                    row_v[g],
                    out_hbm.at[pl.ds(base, C), pl.ds(g * COORD, COORD)])
            return 0

        lax.fori_loop(0, NCHUNK, chunk, 0)

    return body


_sc_kernel = _make_sc_kernel()


def kernel(bbox, x_table, y_table, h_table, w_table):
    bbox_t = jnp.transpose(bbox.reshape(T, 4))
    out = _sc_kernel(bbox_t, x_table, y_table, h_table, w_table)
    return out.reshape(B, N, D_OUT)


# speedup vs baseline: 8.3065x; 1.0903x over previous
import functools

import jax
import jax.numpy as jnp
from jax import lax
from jax.experimental import pallas as pl
from jax.experimental.pallas import tpu as pltpu
from jax.experimental.pallas import tpu_sc as plsc

B = 1024
N = 200
COORD = 128
T = B * N
D_OUT = 6 * COORD
NW = 32
TPW = T // NW
C = 64
NCHUNK = TPW // C
NPAIR = NCHUNK // 2


def _make_sc_kernel():
    mesh = plsc.VectorSubcoreMesh(core_axis_name="c", subcore_axis_name="s")

    @functools.partial(
        pl.kernel,
        out_type=jax.ShapeDtypeStruct((T, D_OUT), jnp.float32),
        mesh=mesh,
        scratch_types=[
            pltpu.VMEM((4, 2 * C), jnp.int32),
            [[pltpu.VMEM((C,), jnp.int32) for _ in range(6)] for _ in range(2)],
            [[pltpu.VMEM((C, COORD), jnp.float32) for _ in range(6)] for _ in range(2)],
            [pltpu.SemaphoreType.DMA for _ in range(2)],
            [pltpu.SemaphoreType.DMA for _ in range(2)],
        ],
    )
    def body(bbox_hbm, x_hbm, y_hbm, h_hbm, w_hbm, out_hbm,
             bb_v, idx_v, row_v, gsem, ssem):
        wid = lax.axis_index("s") * 2 + lax.axis_index("c")
        tables = (x_hbm, y_hbm, x_hbm, y_hbm, h_hbm, w_hbm)

        def fg(ci, s):
            base = wid * TPW + ci * C
            if s == 0:
                pltpu.sync_copy(bbox_hbm.at[:, pl.ds(base, 2 * C)], bb_v)
            for i in range(C // 16):
                sl = pl.ds(i * 16, 16)
                bsl = pl.ds(s * C + i * 16, 16)
                c0 = bb_v[0, bsl]
                c1 = bb_v[1, bsl]
                c2 = bb_v[2, bsl]
                c3 = bb_v[3, bsl]
                idx_v[s][0][sl] = c0
                idx_v[s][1][sl] = c1
                idx_v[s][2][sl] = c2
                idx_v[s][3][sl] = c3
                idx_v[s][4][sl] = c3 - c1
                idx_v[s][5][sl] = c2 - c0
            for g in range(6):
                pltpu.async_copy(tables[g].at[idx_v[s][g]], row_v[s][g], gsem[s])

        def wg(s):
            for g in range(6):
                pltpu.make_async_copy(
                    tables[g].at[idx_v[s][g]], row_v[s][g], gsem[s]).wait()

        def fs(ci, s):
            base = wid * TPW + ci * C
            for g in range(6):
                pltpu.async_copy(
                    row_v[s][g],
                    out_hbm.at[pl.ds(base, C), pl.ds(g * COORD, COORD)],
                    ssem[s])

        def ws(ci, s):
            base = wid * TPW + ci * C
            for g in range(6):
                pltpu.make_async_copy(
                    row_v[s][g],
                    out_hbm.at[pl.ds(base, C), pl.ds(g * COORD, COORD)],
                    ssem[s]).wait()

        fg(0, 0)
        wg(0); fs(0, 0); fg(1, 1)
        wg(1); fs(1, 1); ws(0, 0); fg(2, 0)

        def pair(p, _):
            e = 2 * p
            wg(0); fs(e, 0); ws(e - 1, 1); fg(e + 1, 1)
            wg(1); fs(e + 1, 1); ws(e, 0); fg(e + 2, 0)
            return 0

        lax.fori_loop(1, NPAIR - 1, pair, 0)

        last = NCHUNK - 1
        wg(0); fs(last - 1, 0); ws(last - 2, 1); fg(last, 1)
        wg(1); fs(last, 1)
        ws(last - 1, 0); ws(last, 1)

    return body


_sc_kernel = _make_sc_kernel()


def kernel(bbox, x_table, y_table, h_table, w_table):
    bbox_t = jnp.transpose(bbox.reshape(T, 4))
    out = _sc_kernel(bbox_t, x_table, y_table, h_table, w_table)
    return out.reshape(B, N, D_OUT)


# speedup vs baseline: 10.4041x; 1.2525x over previous
import functools

import jax
import jax.numpy as jnp
from jax import lax
from jax.experimental import pallas as pl
from jax.experimental.pallas import tpu as pltpu
from jax.experimental.pallas import tpu_sc as plsc

B = 1024
N = 200
COORD = 128
T = B * N
D_OUT = 6 * COORD
NW = 32
TPW = T // NW
C = 64
NCHUNK = TPW // C
NPAIR = NCHUNK // 2


def _make_sc_kernel():
    mesh = plsc.VectorSubcoreMesh(core_axis_name="c", subcore_axis_name="s")

    @functools.partial(
        pl.kernel,
        out_type=jax.ShapeDtypeStruct((T, D_OUT), jnp.float32),
        mesh=mesh,
        scratch_types=[
            pltpu.VMEM((4, 2 * C), jnp.int32),
            [[pltpu.VMEM((C,), jnp.int32) for _ in range(6)] for _ in range(2)],
            [[pltpu.VMEM((C, COORD), jnp.float32) for _ in range(6)] for _ in range(2)],
            [pltpu.SemaphoreType.DMA for _ in range(2)],
            [pltpu.SemaphoreType.DMA for _ in range(2)],
            [pltpu.VMEM_SHARED((1024, COORD), jnp.float32) for _ in range(3)],
        ],
    )
    def body(bbox_hbm, x_hbm, y_hbm, h_hbm, w_hbm, out_hbm,
             bb_v, idx_v, row_v, gsem, ssem, sp):
        sid = lax.axis_index("s")
        wid = sid * 2 + lax.axis_index("c")

        for t, src in enumerate((x_hbm, y_hbm, h_hbm)):
            stripe = pl.ds(sid * 64, 64)
            pltpu.sync_copy(src.at[stripe, :], row_v[0][0])
            pltpu.sync_copy(row_v[0][0], sp[t].at[stripe, :])
        plsc.subcore_barrier()

        tables = (x_hbm, y_hbm, x_hbm, y_hbm, h_hbm, w_hbm)
        sp_tables = (sp[0], sp[1], sp[0], sp[1], sp[2], None)

        def fg(ci, s):
            base = wid * TPW + ci * C
            if s == 0:
                pltpu.sync_copy(bbox_hbm.at[:, pl.ds(base, 2 * C)], bb_v)
            for i in range(C // 16):
                sl = pl.ds(i * 16, 16)
                bsl = pl.ds(s * C + i * 16, 16)
                c0 = bb_v[0, bsl]
                c1 = bb_v[1, bsl]
                c2 = bb_v[2, bsl]
                c3 = bb_v[3, bsl]
                idx_v[s][0][sl] = c0
                idx_v[s][1][sl] = c1
                idx_v[s][2][sl] = c2
                idx_v[s][3][sl] = c3
                idx_v[s][4][sl] = c3 - c1
                idx_v[s][5][sl] = c2 - c0
            for g in range(6):
                if sp_tables[g] is not None:
                    pltpu.sync_copy(sp_tables[g].at[idx_v[s][g]], row_v[s][g])
                else:
                    pltpu.async_copy(
                        tables[g].at[idx_v[s][g]], row_v[s][g], gsem[s])

        def wg(s):
            for g in range(6):
                if sp_tables[g] is None:
                    pltpu.make_async_copy(
                        tables[g].at[idx_v[s][g]], row_v[s][g], gsem[s]).wait()

        def fs(ci, s):
            base = wid * TPW + ci * C
            for g in range(6):
                pltpu.async_copy(
                    row_v[s][g],
                    out_hbm.at[pl.ds(base, C), pl.ds(g * COORD, COORD)],
                    ssem[s])

        def ws(ci, s):
            base = wid * TPW + ci * C
            for g in range(6):
                pltpu.make_async_copy(
                    row_v[s][g],
                    out_hbm.at[pl.ds(base, C), pl.ds(g * COORD, COORD)],
                    ssem[s]).wait()

        fg(0, 0)
        wg(0); fs(0, 0); fg(1, 1)
        wg(1); fs(1, 1); ws(0, 0); fg(2, 0)

        def pair(p, _):
            e = 2 * p
            wg(0); fs(e, 0); ws(e - 1, 1); fg(e + 1, 1)
            wg(1); fs(e + 1, 1); ws(e, 0); fg(e + 2, 0)
            return 0

        lax.fori_loop(1, NPAIR - 1, pair, 0)

        last = NCHUNK - 1
        wg(0); fs(last - 1, 0); ws(last - 2, 1); fg(last, 1)
        wg(1); fs(last, 1)
        ws(last - 1, 0); ws(last, 1)

    return body


_sc_kernel = _make_sc_kernel()


def kernel(bbox, x_table, y_table, h_table, w_table):
    bbox_t = jnp.transpose(bbox.reshape(T, 4))
    out = _sc_kernel(bbox_t, x_table, y_table, h_table, w_table)
    return out.reshape(B, N, D_OUT)


# speedup vs baseline: 12.7878x; 1.2291x over previous
import functools

import jax
import jax.numpy as jnp
from jax import lax
from jax.experimental import pallas as pl
from jax.experimental.pallas import tpu as pltpu
from jax.experimental.pallas import tpu_sc as plsc

B = 1024
N = 200
COORD = 128
T = B * N
D_OUT = 6 * COORD
NW = 32
TPW = T // NW
C = 64
NCHUNK = TPW // C
NPAIR = NCHUNK // 2


def _make_sc_kernel():
    mesh = plsc.VectorSubcoreMesh(core_axis_name="c", subcore_axis_name="s")

    @functools.partial(
        pl.kernel,
        out_type=jax.ShapeDtypeStruct((T, D_OUT), jnp.float32),
        mesh=mesh,
        scratch_types=[
            pltpu.VMEM((4, 2 * C), jnp.int32),
            [[pltpu.VMEM((C,), jnp.int32) for _ in range(6)] for _ in range(2)],
            [[pltpu.VMEM((C, COORD), jnp.float32) for _ in range(6)] for _ in range(2)],
            [pltpu.SemaphoreType.DMA for _ in range(2)],
            [pltpu.SemaphoreType.DMA for _ in range(2)],
            [pltpu.VMEM_SHARED((1024, COORD), jnp.float32) for _ in range(3)],
        ],
    )
    def body(bbox_hbm, x_hbm, y_hbm, h_hbm, w_hbm, out_hbm,
             bb_v, idx_v, row_v, gsem, ssem, sp):
        sid = lax.axis_index("s")
        wid = sid * 2 + lax.axis_index("c")

        for t, src in enumerate((x_hbm, y_hbm, h_hbm)):
            stripe = pl.ds(sid * 64, 64)
            pltpu.sync_copy(src.at[stripe, :], row_v[0][0])
            pltpu.sync_copy(row_v[0][0], sp[t].at[stripe, :])
        plsc.subcore_barrier()

        tables = (x_hbm, y_hbm, x_hbm, y_hbm, h_hbm, w_hbm)
        sp_tables = (sp[0], sp[1], sp[0], sp[1], sp[2], None)

        def fg(ci, s):
            base = wid * TPW + ci * C
            if s == 0:
                pltpu.sync_copy(bbox_hbm.at[:, pl.ds(base, 2 * C)], bb_v)
            for i in range(C // 16):
                sl = pl.ds(i * 16, 16)
                bsl = pl.ds(s * C + i * 16, 16)
                c0 = bb_v[0, bsl]
                c1 = bb_v[1, bsl]
                c2 = bb_v[2, bsl]
                c3 = bb_v[3, bsl]
                idx_v[s][0][sl] = c0
                idx_v[s][1][sl] = c1
                idx_v[s][2][sl] = c2
                idx_v[s][3][sl] = c3
                idx_v[s][4][sl] = c3 - c1
                idx_v[s][5][sl] = c2 - c0
            for g in range(6):
                if sp_tables[g] is None:
                    pltpu.async_copy(
                        tables[g].at[idx_v[s][g]], row_v[s][g], gsem[s])
            for g in range(6):
                if sp_tables[g] is not None:
                    pltpu.sync_copy(sp_tables[g].at[idx_v[s][g]], row_v[s][g])

        def wg(s):
            for g in range(6):
                if sp_tables[g] is None:
                    pltpu.make_async_copy(
                        tables[g].at[idx_v[s][g]], row_v[s][g], gsem[s]).wait()

        def fs(ci, s):
            base = wid * TPW + ci * C
            for g in range(6):
                pltpu.async_copy(
                    row_v[s][g],
                    out_hbm.at[pl.ds(base, C), pl.ds(g * COORD, COORD)],
                    ssem[s])

        def ws(ci, s):
            base = wid * TPW + ci * C
            for g in range(6):
                pltpu.make_async_copy(
                    row_v[s][g],
                    out_hbm.at[pl.ds(base, C), pl.ds(g * COORD, COORD)],
                    ssem[s]).wait()

        fg(0, 0)
        wg(0); fs(0, 0); fg(1, 1)
        wg(1); fs(1, 1); ws(0, 0); fg(2, 0)

        def pair(p, _):
            e = 2 * p
            wg(0); fs(e, 0); ws(e - 1, 1); fg(e + 1, 1)
            wg(1); fs(e + 1, 1); ws(e, 0); fg(e + 2, 0)
            return 0

        lax.fori_loop(1, NPAIR - 1, pair, 0)

        last = NCHUNK - 1
        wg(0); fs(last - 1, 0); ws(last - 2, 1); fg(last, 1)
        wg(1); fs(last, 1)
        ws(last - 1, 0); ws(last, 1)

    return body


_sc_kernel = _make_sc_kernel()


def kernel(bbox, x_table, y_table, h_table, w_table):
    bbox_t = jnp.transpose(bbox.reshape(T, 4))
    out = _sc_kernel(bbox_t, x_table, y_table, h_table, w_table)
    return out.reshape(B, N, D_OUT)
